# baseline (speedup 1.0000x reference)
import functools

import jax
import jax.numpy as jnp
from jax import lax
from jax.experimental import pallas as pl
from jax.experimental.pallas import tpu as pltpu
from jax.experimental.pallas import tpu_sc as plsc

_NC = 2
_NS = 16
_NW = _NC * _NS
_L = 16
_D = 64
_HP = 56
_GP = 8


def _tile_sum(rows_ref, st_ref, row, per_row):
    for k in range(_D // _L):
        sl = pl.ds(k * _L, _L)
        acc0 = rows_ref[row * per_row, sl]
        acc1 = rows_ref[row * per_row + 1, sl]
        for j in range(2, per_row, 2):
            acc0 = acc0 + rows_ref[row * per_row + j, sl]
        for j in range(3, per_row, 2):
            acc1 = acc1 + rows_ref[row * per_row + j, sl]
        st_ref[row, sl] = acc0 + acc1


def _make_sc_kernel(B):
    BPW = B // _NW
    UCH = min(128, BPW)
    NCU = BPW // UCH
    NCH = BPW // 2
    GCR = min(16, BPW)
    NCG = BPW // GCR

    mesh = plsc.VectorSubcoreMesh(
        core_axis_name="c", subcore_axis_name="s",
        num_cores=_NC, num_subcores=_NS)
    f32 = jnp.float32
    i32 = jnp.int32
    out64 = jax.ShapeDtypeStruct((B, _D), f32)

    @functools.partial(
        pl.kernel,
        out_type=(out64, out64, out64, out64),
        mesh=mesh,
        compiler_params=pltpu.CompilerParams(use_tc_tiling_on_sc=False),
        scratch_types=[
            pltpu.VMEM((UCH,), i32),
            pltpu.VMEM((UCH, _D), f32),
            pltpu.VMEM((2 * _HP,), i32),
            pltpu.VMEM((2 * _HP,), i32),
            pltpu.VMEM((2 * _HP, _D), f32),
            pltpu.VMEM((2 * _HP, _D), f32),
            pltpu.VMEM((2, _D), f32),
            pltpu.VMEM((2, _D), f32),
            pltpu.VMEM((GCR * _GP,), i32),
            pltpu.VMEM((GCR * _GP,), i32),
            pltpu.VMEM((GCR * _GP, _D), f32),
            pltpu.VMEM((GCR * _GP, _D), f32),
            pltpu.VMEM((GCR, _D), f32),
            pltpu.VMEM((GCR, _D), f32),
            pltpu.SemaphoreType.DMA,
            pltpu.SemaphoreType.DMA,
            pltpu.SemaphoreType.DMA,
            pltpu.SemaphoreType.DMA,
            pltpu.SemaphoreType.DMA,
            pltpu.SemaphoreType.DMA,
            pltpu.SemaphoreType.DMA,
        ],
    )
    def sc_kernel(uidx, iidx, hidx, gidx, utab, itab, gtab,
                  ue, hs, ie, gs,
                  idx_v, emb_v,
                  h_idx0, h_idx1, h_rows0, h_rows1, h_st0, h_st1,
                  g_idx0, g_idx1, g_rows0, g_rows1, g_st0, g_st1,
                  sem_a, sem_g0, sem_g1, sem_i0, sem_i1, sem_o0, sem_o1):
        wid = lax.axis_index("s") * _NC + lax.axis_index("c")
        base = wid * BPW

        for tab, src, dst in ((utab, uidx, ue), (itab, iidx, ie)):
            for c in range(NCU):
                row0 = base + c * UCH
                pltpu.sync_copy(src.at[pl.ds(row0, UCH)], idx_v)
                pltpu.async_copy(tab.at[idx_v], emb_v, sem_a).wait()
                pltpu.sync_copy(emb_v, dst.at[pl.ds(row0, UCH)])

        def pooled_gather(tab, idx_flat, out, per_row, rpc, nch,
                          idx_bufs, row_bufs, st_bufs):
            ipc = per_row * rpc
            ibase = base * per_row
            semg = (sem_g0, sem_g1)
            semi = (sem_i0, sem_i1)
            semo = (sem_o0, sem_o1)

            def idx_src(c):
                return idx_flat.at[pl.ds(ibase + c * ipc, ipc)]

            pltpu.sync_copy(idx_src(0), idx_bufs[0])
            pltpu.async_copy(tab.at[idx_bufs[0]], row_bufs[0], semg[0])
            pltpu.async_copy(idx_src(1), idx_bufs[1], semi[1])

            def step(g, carry):
                for b in (0, 1):
                    ob = 1 - b
                    c = g * 2 + b

                    @pl.when(c + 1 < nch)
                    def _():
                        pltpu.make_async_copy(
                            idx_src(c + 1), idx_bufs[ob], semi[ob]).wait()
                        pltpu.async_copy(
                            tab.at[idx_bufs[ob]], row_bufs[ob], semg[ob])

                    pltpu.make_async_copy(
                        tab.at[idx_bufs[b]], row_bufs[b], semg[b]).wait()

                    @pl.when(c >= 2)
                    def _():
                        pltpu.make_async_copy(
                            st_bufs[b],
                            out.at[pl.ds(base + (c - 2) * rpc, rpc)],
                            semo[b]).wait()

                    for r in range(rpc):
                        _tile_sum(row_bufs[b], st_bufs[b], r, per_row)

                    pltpu.async_copy(
                        st_bufs[b], out.at[pl.ds(base + c * rpc, rpc)],
                        semo[b])

                    @pl.when(c + 2 < nch)
                    def _():
                        pltpu.async_copy(idx_src(c + 2), idx_bufs[b], semi[b])
                return carry

            lax.fori_loop(0, nch // 2, step, 0)
            pltpu.make_async_copy(
                st_bufs[0], out.at[pl.ds(base + (nch - 2) * rpc, rpc)],
                semo[0]).wait()
            pltpu.make_async_copy(
                st_bufs[1], out.at[pl.ds(base + (nch - 1) * rpc, rpc)],
                semo[1]).wait()

        pooled_gather(itab, hidx, hs, _HP, 2, NCH,
                      (h_idx0, h_idx1), (h_rows0, h_rows1), (h_st0, h_st1))
        pooled_gather(gtab, gidx, gs, _GP, GCR, NCG,
                      (g_idx0, g_idx1), (g_rows0, g_rows1), (g_st0, g_st1))

    return sc_kernel


def _tc_towers(hidx, gidx, ue, hs, ie, gs,
               uW1a, uW1b, ub1, uW2, ub2, iW1a, iW1b, ib1, iW2, ib2,
               blk=2048):
    B, hist = hidx.shape
    gen = gidx.shape[1]
    HID = uW1a.shape[1]
    f32 = jnp.float32

    def body(hidx_ref, gidx_ref, ue_ref, hs_ref, ie_ref, gs_ref,
             uW1a_ref, uW1b_ref, ub1_ref, uW2_ref, ub2_ref,
             iW1a_ref, iW1b_ref, ib1_ref, iW2_ref, ib2_ref, out_ref):
        hcnt = jnp.sum((hidx_ref[...] != 0).astype(f32), axis=1, keepdims=True)
        hmean = hs_ref[...] / (hcnt + 1e-8)
        uh = jnp.dot(ue_ref[...], uW1a_ref[...], preferred_element_type=f32)
        uh += jnp.dot(hmean, uW1b_ref[...], preferred_element_type=f32)
        uh = jnp.maximum(uh + ub1_ref[...], 0.0)
        uv = jnp.dot(uh, uW2_ref[...], preferred_element_type=f32) + ub2_ref[...]

        gcnt = jnp.sum((gidx_ref[...] != 0).astype(f32), axis=1, keepdims=True)
        gmean = gs_ref[...] / (gcnt + 1e-8)
        ih = jnp.dot(ie_ref[...], iW1a_ref[...], preferred_element_type=f32)
        ih += jnp.dot(gmean, iW1b_ref[...], preferred_element_type=f32)
        ih = jnp.maximum(ih + ib1_ref[...], 0.0)
        iv = jnp.dot(ih, iW2_ref[...], preferred_element_type=f32) + ib2_ref[...]

        logits = jnp.sum(uv * iv, axis=1)
        out_ref[...] = 1.0 / (1.0 + jnp.exp(-logits))

    grid = B // blk
    row_spec = lambda w: pl.BlockSpec((blk, w), lambda i: (i, 0))
    full_spec = lambda a: pl.BlockSpec(a.shape, lambda i: (0,) * a.ndim)
    return pl.pallas_call(
        body,
        grid=(grid,),
        in_specs=[
            row_spec(hist), row_spec(gen),
            row_spec(_D), row_spec(_D), row_spec(_D), row_spec(_D),
            full_spec(uW1a), full_spec(uW1b), full_spec(ub1),
            full_spec(uW2), full_spec(ub2),
            full_spec(iW1a), full_spec(iW1b), full_spec(ib1),
            full_spec(iW2), full_spec(ib2),
        ],
        out_specs=pl.BlockSpec((blk,), lambda i: (i,)),
        out_shape=jax.ShapeDtypeStruct((B,), f32),
    )(hidx, gidx, ue, hs, ie, gs,
      uW1a, uW1b, ub1, uW2, ub2, iW1a, iW1b, ib1, iW2, ib2)


def kernel(user_indices, history_indices, item_indices, genre_indices,
           item_table, user_table, genre_table,
           uW1, ub1, uW2, ub2, iW1, ib1, iW2, ib2):
    B = user_indices.shape[0]
    hist = history_indices.shape[1]
    gen = genre_indices.shape[1]
    i32 = jnp.int32

    hflat = jnp.concatenate(
        [history_indices.astype(i32),
         jnp.zeros((B, _HP - hist), i32)], axis=1).reshape(-1)
    gflat = jnp.concatenate(
        [genre_indices.astype(i32),
         jnp.zeros((B, _GP - gen), i32)], axis=1).reshape(-1)

    ue, hs, ie, gs = _make_sc_kernel(B)(
        user_indices.astype(i32), item_indices.astype(i32), hflat, gflat,
        user_table, item_table, genre_table)

    return _tc_towers(
        history_indices.astype(i32), genre_indices.astype(i32),
        ue, hs, ie, gs,
        uW1[:_D], uW1[_D:], ub1.reshape(1, -1), uW2, ub2.reshape(1, -1),
        iW1[:_D], iW1[_D:], ib1.reshape(1, -1), iW2, ib2.reshape(1, -1))


# speedup vs baseline: 1.0090x; 1.0090x over previous
import functools

import jax
import jax.numpy as jnp
from jax import lax
from jax.experimental import pallas as pl
from jax.experimental.pallas import tpu as pltpu
from jax.experimental.pallas import tpu_sc as plsc

_NC = 2
_NS = 16
_NW = _NC * _NS
_L = 16
_D = 64
_HP = 56
_GP = 8


def _tile_sum(rows_ref, st_ref, row, per_row, roff):
    for k in range(_D // _L):
        sl = pl.ds(k * _L, _L)
        acc0 = rows_ref[roff, sl]
        acc1 = rows_ref[roff + 1, sl]
        for j in range(2, per_row, 2):
            acc0 = acc0 + rows_ref[roff + j, sl]
        for j in range(3, per_row, 2):
            acc1 = acc1 + rows_ref[roff + j, sl]
        st_ref[row, sl] = acc0 + acc1


def _make_sc_kernel(B):
    BPW = B // _NW
    UCH = min(512, BPW)
    NCU = BPW // UCH
    HCR = min(8, BPW)
    NCH = BPW // HCR
    GCR = min(64, BPW)
    NCG = BPW // GCR
    RB = max(UCH, HCR * _HP, GCR * _GP)

    mesh = plsc.VectorSubcoreMesh(
        core_axis_name="c", subcore_axis_name="s",
        num_cores=_NC, num_subcores=_NS)
    f32 = jnp.float32
    i32 = jnp.int32
    out64 = jax.ShapeDtypeStruct((B, _D), f32)

    @functools.partial(
        pl.kernel,
        out_type=(out64, out64, out64, out64),
        mesh=mesh,
        compiler_params=pltpu.CompilerParams(use_tc_tiling_on_sc=False),
        scratch_types=[
            pltpu.VMEM((RB,), i32),
            pltpu.VMEM((RB,), i32),
            pltpu.VMEM((RB, _D), f32),
            pltpu.VMEM((RB, _D), f32),
            pltpu.VMEM((HCR, _D), f32),
            pltpu.VMEM((HCR, _D), f32),
            pltpu.VMEM((GCR, _D), f32),
            pltpu.VMEM((GCR, _D), f32),
            pltpu.SemaphoreType.DMA,
            pltpu.SemaphoreType.DMA,
            pltpu.SemaphoreType.DMA,
            pltpu.SemaphoreType.DMA,
            pltpu.SemaphoreType.DMA,
            pltpu.SemaphoreType.DMA,
            pltpu.SemaphoreType.DMA,
        ],
    )
    def sc_kernel(uidx, iidx, hidx, gidx, utab, itab, gtab,
                  ue, hs, ie, gs,
                  idx_b0, idx_b1, rows_b0, rows_b1,
                  h_st0, h_st1, g_st0, g_st1,
                  sem_a, sem_g0, sem_g1, sem_i0, sem_i1, sem_o0, sem_o1):
        wid = lax.axis_index("s") * _NC + lax.axis_index("c")
        base = wid * BPW

        for tab, src, dst in ((utab, uidx, ue), (itab, iidx, ie)):
            for c in range(NCU):
                row0 = base + c * UCH
                idx_v = idx_b0.at[pl.ds(0, UCH)]
                emb_v = rows_b0.at[pl.ds(0, UCH)]
                pltpu.sync_copy(src.at[pl.ds(row0, UCH)], idx_v)
                pltpu.async_copy(tab.at[idx_v], emb_v, sem_a).wait()
                pltpu.sync_copy(emb_v, dst.at[pl.ds(row0, UCH)])

        def pooled_gather(tab, idx_flat, out, per_row, rpc, nch, st_bufs):
            ipc = per_row * rpc
            ibase = base * per_row
            idx_bufs = (idx_b0.at[pl.ds(0, ipc)], idx_b1.at[pl.ds(0, ipc)])
            row_bufs = (rows_b0.at[pl.ds(0, ipc)], rows_b1.at[pl.ds(0, ipc)])
            semg = (sem_g0, sem_g1)
            semi = (sem_i0, sem_i1)
            semo = (sem_o0, sem_o1)

            def idx_src(c):
                return idx_flat.at[pl.ds(ibase + c * ipc, ipc)]

            pltpu.sync_copy(idx_src(0), idx_bufs[0])
            pltpu.async_copy(tab.at[idx_bufs[0]], row_bufs[0], semg[0])
            pltpu.async_copy(idx_src(1), idx_bufs[1], semi[1])

            def step(g, carry):
                for b in (0, 1):
                    ob = 1 - b
                    c = g * 2 + b

                    @pl.when(c + 1 < nch)
                    def _():
                        pltpu.make_async_copy(
                            idx_src(c + 1), idx_bufs[ob], semi[ob]).wait()
                        pltpu.async_copy(
                            tab.at[idx_bufs[ob]], row_bufs[ob], semg[ob])

                    pltpu.make_async_copy(
                        tab.at[idx_bufs[b]], row_bufs[b], semg[b]).wait()

                    @pl.when(c >= 2)
                    def _():
                        pltpu.make_async_copy(
                            st_bufs[b],
                            out.at[pl.ds(base + (c - 2) * rpc, rpc)],
                            semo[b]).wait()

                    rows_full = rows_b0 if b == 0 else rows_b1

                    def sum_row(r, carry2):
                        _tile_sum(rows_full, st_bufs[b], r, per_row,
                                  r * per_row)
                        return carry2

                    lax.fori_loop(0, rpc, sum_row, 0)

                    pltpu.async_copy(
                        st_bufs[b], out.at[pl.ds(base + c * rpc, rpc)],
                        semo[b])

                    @pl.when(c + 2 < nch)
                    def _():
                        pltpu.async_copy(idx_src(c + 2), idx_bufs[b], semi[b])
                return carry

            lax.fori_loop(0, nch // 2, step, 0)
            pltpu.make_async_copy(
                st_bufs[0], out.at[pl.ds(base + (nch - 2) * rpc, rpc)],
                semo[0]).wait()
            pltpu.make_async_copy(
                st_bufs[1], out.at[pl.ds(base + (nch - 1) * rpc, rpc)],
                semo[1]).wait()

        pooled_gather(itab, hidx, hs, _HP, HCR, NCH, (h_st0, h_st1))
        pooled_gather(gtab, gidx, gs, _GP, GCR, NCG, (g_st0, g_st1))

    return sc_kernel


def _tc_towers(hidx, gidx, ue, hs, ie, gs,
               uW1a, uW1b, ub1, uW2, ub2, iW1a, iW1b, ib1, iW2, ib2,
               blk=2048):
    B, hist = hidx.shape
    gen = gidx.shape[1]
    HID = uW1a.shape[1]
    f32 = jnp.float32

    def body(hidx_ref, gidx_ref, ue_ref, hs_ref, ie_ref, gs_ref,
             uW1a_ref, uW1b_ref, ub1_ref, uW2_ref, ub2_ref,
             iW1a_ref, iW1b_ref, ib1_ref, iW2_ref, ib2_ref, out_ref):
        hcnt = jnp.sum((hidx_ref[...] != 0).astype(f32), axis=1, keepdims=True)
        hmean = hs_ref[...] / (hcnt + 1e-8)
        uh = jnp.dot(ue_ref[...], uW1a_ref[...], preferred_element_type=f32)
        uh += jnp.dot(hmean, uW1b_ref[...], preferred_element_type=f32)
        uh = jnp.maximum(uh + ub1_ref[...], 0.0)
        uv = jnp.dot(uh, uW2_ref[...], preferred_element_type=f32) + ub2_ref[...]

        gcnt = jnp.sum((gidx_ref[...] != 0).astype(f32), axis=1, keepdims=True)
        gmean = gs_ref[...] / (gcnt + 1e-8)
        ih = jnp.dot(ie_ref[...], iW1a_ref[...], preferred_element_type=f32)
        ih += jnp.dot(gmean, iW1b_ref[...], preferred_element_type=f32)
        ih = jnp.maximum(ih + ib1_ref[...], 0.0)
        iv = jnp.dot(ih, iW2_ref[...], preferred_element_type=f32) + ib2_ref[...]

        logits = jnp.sum(uv * iv, axis=1)
        out_ref[...] = 1.0 / (1.0 + jnp.exp(-logits))

    grid = B // blk
    row_spec = lambda w: pl.BlockSpec((blk, w), lambda i: (i, 0))
    full_spec = lambda a: pl.BlockSpec(a.shape, lambda i: (0,) * a.ndim)
    return pl.pallas_call(
        body,
        grid=(grid,),
        in_specs=[
            row_spec(hist), row_spec(gen),
            row_spec(_D), row_spec(_D), row_spec(_D), row_spec(_D),
            full_spec(uW1a), full_spec(uW1b), full_spec(ub1),
            full_spec(uW2), full_spec(ub2),
            full_spec(iW1a), full_spec(iW1b), full_spec(ib1),
            full_spec(iW2), full_spec(ib2),
        ],
        out_specs=pl.BlockSpec((blk,), lambda i: (i,)),
        out_shape=jax.ShapeDtypeStruct((B,), f32),
    )(hidx, gidx, ue, hs, ie, gs,
      uW1a, uW1b, ub1, uW2, ub2, iW1a, iW1b, ib1, iW2, ib2)


def kernel(user_indices, history_indices, item_indices, genre_indices,
           item_table, user_table, genre_table,
           uW1, ub1, uW2, ub2, iW1, ib1, iW2, ib2):
    B = user_indices.shape[0]
    hist = history_indices.shape[1]
    gen = genre_indices.shape[1]
    i32 = jnp.int32

    hflat = jnp.concatenate(
        [history_indices.astype(i32),
         jnp.zeros((B, _HP - hist), i32)], axis=1).reshape(-1)
    gflat = jnp.concatenate(
        [genre_indices.astype(i32),
         jnp.zeros((B, _GP - gen), i32)], axis=1).reshape(-1)

    ue, hs, ie, gs = _make_sc_kernel(B)(
        user_indices.astype(i32), item_indices.astype(i32), hflat, gflat,
        user_table, item_table, genre_table)

    return _tc_towers(
        history_indices.astype(i32), genre_indices.astype(i32),
        ue, hs, ie, gs,
        uW1[:_D], uW1[_D:], ub1.reshape(1, -1), uW2, ub2.reshape(1, -1),
        iW1[:_D], iW1[_D:], ib1.reshape(1, -1), iW2, ib2.reshape(1, -1))
